# baseline (speedup 1.0000x reference)
import functools

import jax
import jax.numpy as jnp
from jax import lax
from jax.experimental import pallas as pl
from jax.experimental.pallas import tpu as pltpu
from jax.experimental.pallas import tpu_sc as plsc

B = 16384
D = 128
NC = 2
NS = 16
NW = NC * NS
BW = B // NW
C = 32
NBUF = 8
NCHUNK = BW // C
G16 = C // 16
IPR = 128 // C


def _mf_body(uid_hbm, iid_hbm, uemb_hbm, iemb_hbm, out_hbm,
             uid_v, iid_v, ubuf, vbuf, stage, out_v, sem_uv):
    wid = lax.axis_index("s") * NC + lax.axis_index("c")

    pltpu.sync_copy(uid_hbm.at[wid], uid_v)
    pltpu.sync_copy(iid_hbm.at[wid], iid_v)

    def chunk_ids(ids_v, g):
        return ids_v.at[lax.div(g, IPR), pl.ds(lax.rem(g, IPR) * C, C)]

    def start_chunk(g):
        p = lax.rem(g, NBUF)
        pltpu.async_copy(uemb_hbm.at[chunk_ids(uid_v, g)], ubuf.at[p],
                         sem_uv.at[p, 0])
        pltpu.async_copy(iemb_hbm.at[chunk_ids(iid_v, g)], vbuf.at[p],
                         sem_uv.at[p, 1])

    def wait_chunk(g):
        p = lax.rem(g, NBUF)
        pltpu.make_async_copy(uemb_hbm.at[chunk_ids(uid_v, g)], ubuf.at[p],
                              sem_uv.at[p, 0]).wait()
        pltpu.make_async_copy(iemb_hbm.at[chunk_ids(iid_v, g)], vbuf.at[p],
                              sem_uv.at[p, 1]).wait()

    for _g in range(NBUF - 1):
        start_chunk(jnp.int32(_g))

    iota16 = lax.iota(jnp.int32, 16)

    def chunk_body(g, carry):
        @pl.when(g + NBUF - 1 < NCHUNK)
        def _():
            start_chunk(g + NBUF - 1)
        wait_chunk(g)
        p = lax.rem(g, NBUF)

        def group(t, gcarry):
            r0 = t * 16

            def row(j, rcarry):
                for h in range(2):
                    r = r0 + 2 * j + h
                    acc = ubuf[p, r, 0:16] * vbuf[p, r, 0:16]
                    for k in range(1, 8):
                        acc = acc + (ubuf[p, r, k * 16:(k + 1) * 16] *
                                     vbuf[p, r, k * 16:(k + 1) * 16])
                    stage[pl.ds((2 * j + h) * 16, 16)] = acc
                return rcarry
            lax.fori_loop(0, 8, row, 0)

            tot = plsc.load_gather(stage, [iota16 * 16])
            for j in range(1, 16):
                tot = tot + plsc.load_gather(stage, [iota16 * 16 + j])
            out_v[pl.ds(g * C + r0, 16)] = tot
            return gcarry
        lax.fori_loop(0, G16, group, 0)
        return carry

    lax.fori_loop(0, NCHUNK, chunk_body, 0)

    pltpu.sync_copy(out_v, out_hbm.at[pl.ds(wid * BW, BW)])


_mf_kernel = functools.partial(
    pl.kernel,
    out_type=jax.ShapeDtypeStruct((B,), jnp.float32),
    mesh=plsc.VectorSubcoreMesh(core_axis_name="c", subcore_axis_name="s",
                                num_cores=NC, num_subcores=NS),
    compiler_params=pltpu.CompilerParams(needs_layout_passes=False),
    scratch_types=[
        pltpu.VMEM((NCHUNK // IPR, IPR * C), jnp.int32),
        pltpu.VMEM((NCHUNK // IPR, IPR * C), jnp.int32),
        pltpu.VMEM((NBUF, C, D), jnp.float32),
        pltpu.VMEM((NBUF, C, D), jnp.float32),
        pltpu.VMEM((256,), jnp.float32),
        pltpu.VMEM((BW,), jnp.float32),
        pltpu.SemaphoreType.DMA((NBUF, 2)),
    ],
)(_mf_body)


def kernel(user_ids, item_ids, user_emb, item_emb, user_bias, item_bias,
           global_bias):
    uid = user_ids.astype(jnp.int32).reshape(NW, NCHUNK // IPR, IPR * C)
    iid = item_ids.astype(jnp.int32).reshape(NW, NCHUNK // IPR, IPR * C)
    del user_bias, item_bias, global_bias
    return _mf_kernel(uid, iid, user_emb, item_emb)


# speedup vs baseline: 1.0128x; 1.0128x over previous
import functools

import jax
import jax.numpy as jnp
from jax import lax
from jax.experimental import pallas as pl
from jax.experimental.pallas import tpu as pltpu
from jax.experimental.pallas import tpu_sc as plsc

B = 16384
D = 128
NC = 2
NS = 16
NW = NC * NS
BW = B // NW
C = 32
NBUF = 8
NCHUNK = BW // C
G16 = C // 16
IPR = 128 // C


def _mf_body(uid_hbm, iid_hbm, uemb_hbm, iemb_hbm, out_hbm,
             uid_v, iid_v, ubuf, vbuf, stage, out_v, sem_uv):
    wid = lax.axis_index("s") * NC + lax.axis_index("c")

    pltpu.sync_copy(uid_hbm.at[wid], uid_v)
    pltpu.sync_copy(iid_hbm.at[wid], iid_v)

    def chunk_ids(ids_v, g):
        return ids_v.at[lax.div(g, IPR), pl.ds(lax.rem(g, IPR) * C, C)]

    def start_chunk(g):
        p = lax.rem(g, NBUF)
        pltpu.async_copy(uemb_hbm.at[chunk_ids(uid_v, g)], ubuf.at[p],
                         sem_uv.at[p, 0])
        pltpu.async_copy(iemb_hbm.at[chunk_ids(iid_v, g)], vbuf.at[p],
                         sem_uv.at[p, 1])

    def wait_chunk(g):
        p = lax.rem(g, NBUF)
        pltpu.make_async_copy(uemb_hbm.at[chunk_ids(uid_v, g)], ubuf.at[p],
                              sem_uv.at[p, 0]).wait()
        pltpu.make_async_copy(iemb_hbm.at[chunk_ids(iid_v, g)], vbuf.at[p],
                              sem_uv.at[p, 1]).wait()

    for _g in range(NBUF - 1):
        start_chunk(jnp.int32(_g))

    iota16 = lax.iota(jnp.int32, 16)

    def chunk_body(g, carry):
        @pl.when(g + NBUF - 1 < NCHUNK)
        def _():
            start_chunk(g + NBUF - 1)
        wait_chunk(g)
        p = lax.rem(g, NBUF)

        def group(t, gcarry):
            r0 = t * 16

            def row(j, rcarry):
                r = r0 + j
                acc = ubuf[p, r, 0:16] * vbuf[p, r, 0:16]
                for k in range(1, 8):
                    acc = acc + (ubuf[p, r, k * 16:(k + 1) * 16] *
                                 vbuf[p, r, k * 16:(k + 1) * 16])
                stage[pl.ds(j * 16, 16)] = acc
                return rcarry
            lax.fori_loop(0, 16, row, 0)

            tot = plsc.load_gather(stage, [iota16 * 16])
            for j in range(1, 16):
                tot = tot + plsc.load_gather(stage, [iota16 * 16 + j])
            out_v[pl.ds(g * C + r0, 16)] = tot
            return gcarry
        lax.fori_loop(0, G16, group, 0)
        return carry

    lax.fori_loop(0, NCHUNK, chunk_body, 0)

    pltpu.sync_copy(out_v, out_hbm.at[pl.ds(wid * BW, BW)])


_mf_kernel = functools.partial(
    pl.kernel,
    out_type=jax.ShapeDtypeStruct((B,), jnp.float32),
    mesh=plsc.VectorSubcoreMesh(core_axis_name="c", subcore_axis_name="s",
                                num_cores=NC, num_subcores=NS),
    compiler_params=pltpu.CompilerParams(needs_layout_passes=False),
    scratch_types=[
        pltpu.VMEM((NCHUNK // IPR, IPR * C), jnp.int32),
        pltpu.VMEM((NCHUNK // IPR, IPR * C), jnp.int32),
        pltpu.VMEM((NBUF, C, D), jnp.float32),
        pltpu.VMEM((NBUF, C, D), jnp.float32),
        pltpu.VMEM((256,), jnp.float32),
        pltpu.VMEM((BW,), jnp.float32),
        pltpu.SemaphoreType.DMA((NBUF, 2)),
    ],
)(_mf_body)


def kernel(user_ids, item_ids, user_emb, item_emb, user_bias, item_bias,
           global_bias):
    uid = user_ids.astype(jnp.int32).reshape(NW, NCHUNK // IPR, IPR * C)
    iid = item_ids.astype(jnp.int32).reshape(NW, NCHUNK // IPR, IPR * C)
    del user_bias, item_bias, global_bias
    return _mf_kernel(uid, iid, user_emb, item_emb)
